# Initial kernel scaffold; baseline (speedup 1.0000x reference)
import jax
import jax.numpy as jnp
from jax.experimental import pallas as pl


def kernel(x, gates0, choices0, gates1, choices1):
    raise NotImplementedError("write your pallas kernel here")


# speedup vs baseline: 3965.8694x; 3965.8694x over previous
import functools

import jax
import jax.numpy as jnp
from jax.experimental import pallas as pl
from jax.experimental.pallas import tpu as pltpu


def _pad8(a):
    rows = a.shape[0]
    return jnp.pad(a, ((0, 8 - rows), (0, 0)))


def _gator_body(x_ref, g0_ref, c0_ref, g1_ref, c1_ref, out_ref, *, W, G):
    xb = x_ref[...]
    xbb = xb.astype(jnp.bfloat16)

    iota = jax.lax.broadcasted_iota(jnp.int32, (W, G), 0)

    a0 = c0_ref[0:1, :]
    b0 = c0_ref[1:2, :]
    R0 = ((iota == a0).astype(jnp.bfloat16) * 2 + (iota == b0).astype(jnp.bfloat16))
    a1 = c1_ref[0:1, :]
    b1 = c1_ref[1:2, :]
    R1a = ((iota == a1).astype(jnp.bfloat16) * 2 + (iota == b1).astype(jnp.bfloat16))
    R1b = ((iota == (a1 - W)).astype(jnp.bfloat16) * 2
           + (iota == (b1 - W)).astype(jnp.bfloat16))

    Rcat = jnp.concatenate([R0, R1a], axis=1)
    M = jnp.dot(xbb, Rcat, preferred_element_type=jnp.float32)
    t0 = M[:, :G]
    p1 = M[:, G:]

    out0 = jnp.where(t0 < 0.5, g0_ref[0:1, :],
            jnp.where(t0 < 1.5, g0_ref[1:2, :],
             jnp.where(t0 < 2.5, g0_ref[2:3, :], g0_ref[3:4, :])))

    t1 = p1 + jnp.dot(out0.astype(jnp.bfloat16), R1b,
                      preferred_element_type=jnp.float32)

    out1 = jnp.where(t1 < 0.5, g1_ref[0:1, :],
            jnp.where(t1 < 1.5, g1_ref[1:2, :],
             jnp.where(t1 < 2.5, g1_ref[2:3, :], g1_ref[3:4, :])))

    out_ref[:, :W] = xb
    out_ref[:, W:W + G] = out0
    out_ref[:, W + G:] = out1


@jax.jit
def kernel(x, gates0, choices0, gates1, choices1):
    B, W = x.shape
    G = gates0.shape[0]
    TB = 1024

    g0t = _pad8(gates0.T)
    g1t = _pad8(gates1.T)
    c0t = _pad8(choices0.T.astype(jnp.int32))
    c1t = _pad8(choices1.T.astype(jnp.int32))

    body = functools.partial(_gator_body, W=W, G=G)
    out = pl.pallas_call(
        body,
        grid=(B // TB,),
        in_specs=[
            pl.BlockSpec((TB, W), lambda i: (i, 0)),
            pl.BlockSpec((8, G), lambda i: (0, 0)),
            pl.BlockSpec((8, G), lambda i: (0, 0)),
            pl.BlockSpec((8, G), lambda i: (0, 0)),
            pl.BlockSpec((8, G), lambda i: (0, 0)),
        ],
        out_specs=pl.BlockSpec((TB, W + 2 * G), lambda i: (i, 0)),
        out_shape=jax.ShapeDtypeStruct((B, W + 2 * G), jnp.float32),
        compiler_params=pltpu.CompilerParams(
            dimension_semantics=("parallel",),
        ),
    )(x, g0t, c0t, g1t, c1t)
    return out


# speedup vs baseline: 4096.0162x; 1.0328x over previous
import functools

import jax
import jax.numpy as jnp
from jax.experimental import pallas as pl
from jax.experimental.pallas import tpu as pltpu


def _pad8(a):
    rows = a.shape[0]
    return jnp.pad(a, ((0, 8 - rows), (0, 0)))


def _gator_body(x_ref, g0_ref, c0_ref, g1_ref, c1_ref, out_ref, *, W, G):
    xb = x_ref[...]
    xbb = xb.astype(jnp.bfloat16)

    iota = jax.lax.broadcasted_iota(jnp.int32, (W, G), 0)

    a0 = c0_ref[0:1, :]
    b0 = c0_ref[1:2, :]
    R0 = ((iota == a0).astype(jnp.bfloat16) * 2 + (iota == b0).astype(jnp.bfloat16))
    a1 = c1_ref[0:1, :]
    b1 = c1_ref[1:2, :]
    R1a = ((iota == a1).astype(jnp.bfloat16) * 2 + (iota == b1).astype(jnp.bfloat16))
    R1b = ((iota == (a1 - W)).astype(jnp.bfloat16) * 2
           + (iota == (b1 - W)).astype(jnp.bfloat16))

    Rcat = jnp.concatenate([R0, R1a], axis=1)
    M = jnp.dot(xbb, Rcat, preferred_element_type=jnp.float32)
    t0 = M[:, :G]
    p1 = M[:, G:]

    out0 = jnp.where(t0 < 0.5, g0_ref[0:1, :],
            jnp.where(t0 < 1.5, g0_ref[1:2, :],
             jnp.where(t0 < 2.5, g0_ref[2:3, :], g0_ref[3:4, :])))

    t1 = p1 + jnp.dot(out0.astype(jnp.bfloat16), R1b,
                      preferred_element_type=jnp.float32)

    out1 = jnp.where(t1 < 0.5, g1_ref[0:1, :],
            jnp.where(t1 < 1.5, g1_ref[1:2, :],
             jnp.where(t1 < 2.5, g1_ref[2:3, :], g1_ref[3:4, :])))

    out_ref[:, :W] = xb
    out_ref[:, W:W + G] = out0
    out_ref[:, W + G:] = out1


@jax.jit
def kernel(x, gates0, choices0, gates1, choices1):
    B, W = x.shape
    G = gates0.shape[0]
    TB = 2048

    g0t = _pad8(gates0.T)
    g1t = _pad8(gates1.T)
    c0t = _pad8(choices0.T.astype(jnp.int32))
    c1t = _pad8(choices1.T.astype(jnp.int32))

    body = functools.partial(_gator_body, W=W, G=G)
    out = pl.pallas_call(
        body,
        grid=(B // TB,),
        in_specs=[
            pl.BlockSpec((TB, W), lambda i: (i, 0)),
            pl.BlockSpec((8, G), lambda i: (0, 0)),
            pl.BlockSpec((8, G), lambda i: (0, 0)),
            pl.BlockSpec((8, G), lambda i: (0, 0)),
            pl.BlockSpec((8, G), lambda i: (0, 0)),
        ],
        out_specs=pl.BlockSpec((TB, W + 2 * G), lambda i: (i, 0)),
        out_shape=jax.ShapeDtypeStruct((B, W + 2 * G), jnp.float32),
        compiler_params=pltpu.CompilerParams(
            dimension_semantics=("parallel",),
        ),
    )(x, g0t, c0t, g1t, c1t)
    return out
